# baseline (speedup 1.0000x reference)
import functools

import jax
import jax.numpy as jnp
from jax import lax
from jax.experimental import pallas as pl
from jax.experimental.pallas import tpu as pltpu
from jax.experimental.pallas import tpu_sc as plsc

N = 10000
E = 320000
D = 128

NC = 2
NS = 16
NW = NC * NS
C = 80
G = 8
IB = 16
CH = G * IB
EWP = CH * C
EP = NW * EWP
NP = 10240
RPS = NP // NS


def _sc_agg_body(const_rows, *refs):
    if const_rows:
        (ones_cd, dstr, z2d, out_agg, dst_v, rows_v, rows_v1, sh_agg) = refs
    else:
        (feat, srcr, dstr, z2d, out_agg,
         src_v, dst_v, rows_v, rows_v1, sh_agg) = refs
    bufs = (rows_v, rows_v1)

    cid = lax.axis_index("c")
    sid = lax.axis_index("s")
    wid = sid * NC + cid

    pltpu.sync_copy(z2d, rows_v)

    def zero_blk(j, carry):
        pltpu.sync_copy(rows_v, sh_agg.at[pl.ds(sid * RPS + j * C, C)])
        return carry

    lax.fori_loop(0, RPS // C, zero_blk, 0)
    if const_rows:
        pltpu.sync_copy(ones_cd, rows_v)
    plsc.subcore_barrier()

    def pipeline(sem_g, sem_s):
        def superchunk(g, carry):
            if not const_rows:
                pltpu.sync_copy(srcr.at[wid, g], src_v)
            pltpu.sync_copy(dstr.at[wid, g], dst_v)

            if const_rows:
                cps = [
                    pltpu.async_copy(
                        rows_v, sh_agg.at[dst_v.at[k]], sem_s, add=True)
                    for k in range(IB)
                ]
                for cp in cps:
                    cp.wait()
                return carry

            gathers = [None] * IB
            scatters = [None] * IB
            gathers[0] = pltpu.async_copy(
                feat.at[src_v.at[0]], bufs[0], sem_g)
            for k in range(IB):
                buf = bufs[k % 2]
                gathers[k].wait()
                if k >= 1:
                    scatters[k - 1].wait()
                if k + 1 < IB:
                    gathers[k + 1] = pltpu.async_copy(
                        feat.at[src_v.at[k + 1]], bufs[(k + 1) % 2], sem_g)
                scatters[k] = pltpu.async_copy(
                    buf, sh_agg.at[dst_v.at[k]], sem_s, add=True)
            scatters[IB - 1].wait()
            return carry

        lax.fori_loop(0, G, superchunk, 0)

    pl.run_scoped(pipeline, pltpu.SemaphoreType.DMA, pltpu.SemaphoreType.DMA)
    plsc.subcore_barrier()

    def emit_blk(j, carry):
        base = sid * RPS + j * C
        pltpu.sync_copy(sh_agg.at[pl.ds(base, C)], rows_v)
        pltpu.sync_copy(rows_v, out_agg.at[cid, pl.ds(base, C)])
        return carry

    lax.fori_loop(0, RPS // C, emit_blk, 0)


def _make_sc_agg(const_rows):
    scratch = []
    if not const_rows:
        scratch.append(pltpu.VMEM((IB, C), jnp.int32))
    scratch.extend([
        pltpu.VMEM((IB, C), jnp.int32),
        pltpu.VMEM((C, D), jnp.float32),
        pltpu.VMEM((C, D), jnp.float32),
        pltpu.VMEM_SHARED((NP, D), jnp.float32),
    ])
    return pl.kernel(
        functools.partial(_sc_agg_body, const_rows),
        out_type=[jax.ShapeDtypeStruct((NC, NP, D), jnp.float32)],
        mesh=plsc.VectorSubcoreMesh(core_axis_name="c", subcore_axis_name="s"),
        scratch_types=scratch,
    )


def _tc_layer_body(x_ref, p_ref, d_ref, w_ref, b_ref, o_ref):
    deg = d_ref[0, :, :1] + d_ref[1, :, :1]
    inv = 1.0 / jnp.maximum(deg, 1.0)
    agg = (p_ref[0] + p_ref[1]) * inv
    acc = jnp.dot(x_ref[...], w_ref[:D], preferred_element_type=jnp.float32)
    acc = acc + jnp.dot(agg, w_ref[D:], preferred_element_type=jnp.float32)
    o_ref[...] = jnp.maximum(acc + b_ref[...], 0.0)


def _tc_layer(x, parts, degp, W, b2d, blk=2000):
    grid = (N // blk,)
    return pl.pallas_call(
        _tc_layer_body,
        grid=grid,
        in_specs=[
            pl.BlockSpec((blk, D), lambda i: (i, 0)),
            pl.BlockSpec((NC, blk, D), lambda i: (0, i, 0)),
            pl.BlockSpec((NC, blk, D), lambda i: (0, i, 0)),
            pl.BlockSpec((2 * D, D), lambda i: (0, 0)),
            pl.BlockSpec((1, D), lambda i: (0, 0)),
        ],
        out_specs=pl.BlockSpec((blk, D), lambda i: (i, 0)),
        out_shape=jax.ShapeDtypeStruct((N, D), jnp.float32),
    )(x, parts, degp, W, b2d)


_sc_agg = _make_sc_agg(False)
_sc_deg = _make_sc_agg(True)


@jax.jit
def kernel(x, edge_index, W1, b1, W2, b2):
    src = jnp.pad(edge_index[0], (0, EP - E)).reshape(NW, G, IB, C)
    dst = jnp.pad(edge_index[1], (0, EP - E),
                  constant_values=NP - 1).reshape(NW, G, IB, C)
    z2d = jnp.zeros((C, D), jnp.float32)
    ones_cd = jnp.ones((C, D), jnp.float32)

    (degp,) = _sc_deg(ones_cd, dst, z2d)
    (parts1,) = _sc_agg(x, src, dst, z2d)
    h = _tc_layer(x, parts1, degp, W1, b1.reshape(1, D))
    (parts2,) = _sc_agg(h, src, dst, z2d)
    return _tc_layer(h, parts2, degp, W2, b2.reshape(1, D))


# speedup vs baseline: 2.4927x; 2.4927x over previous
import functools

import jax
import jax.numpy as jnp
from jax import lax
from jax.experimental import pallas as pl
from jax.experimental.pallas import tpu as pltpu
from jax.experimental.pallas import tpu_sc as plsc

N = 10000
E = 320000
D = 128

NC = 2
NS = 16
NW = NC * NS
C = 80
G = 5
IB = 25
CH = G * IB
NP = 10240
RPS = NP // NS


def _sc_agg_body(const_rows, *refs):
    if const_rows:
        (ones_cd, dstr, z2d, out_agg, dst_v, rows_v, rows_v1, sh_agg) = refs
    else:
        (feat, srcr, dstr, z2d, out_agg,
         src_v, dst_v, rows_v, rows_v1, sh_agg) = refs
    bufs = (rows_v, rows_v1)

    cid = lax.axis_index("c")
    sid = lax.axis_index("s")
    wid = sid * NC + cid

    pltpu.sync_copy(z2d, rows_v)

    def zero_blk(j, carry):
        pltpu.sync_copy(rows_v, sh_agg.at[pl.ds(sid * RPS + j * C, C)])
        return carry

    lax.fori_loop(0, RPS // C, zero_blk, 0)
    if const_rows:
        pltpu.sync_copy(ones_cd, rows_v)
    plsc.subcore_barrier()

    def pipeline(sem_g, sem_s):
        def superchunk(g, carry):
            if not const_rows:
                pltpu.sync_copy(srcr.at[wid, g], src_v)
            pltpu.sync_copy(dstr.at[wid, g], dst_v)

            if const_rows:
                cps = [
                    pltpu.async_copy(
                        rows_v, sh_agg.at[dst_v.at[k]], sem_s, add=True)
                    for k in range(IB)
                ]
                for cp in cps:
                    cp.wait()
                return carry

            gathers = [None] * IB
            scatters = [None] * IB
            gathers[0] = pltpu.async_copy(
                feat.at[src_v.at[0]], bufs[0], sem_g)
            for k in range(IB):
                buf = bufs[k % 2]
                gathers[k].wait()
                if k >= 1:
                    scatters[k - 1].wait()
                if k + 1 < IB:
                    gathers[k + 1] = pltpu.async_copy(
                        feat.at[src_v.at[k + 1]], bufs[(k + 1) % 2], sem_g)
                scatters[k] = pltpu.async_copy(
                    buf, sh_agg.at[dst_v.at[k]], sem_s, add=True)
            scatters[IB - 1].wait()
            return carry

        lax.fori_loop(0, G, superchunk, 0)

    pl.run_scoped(pipeline, pltpu.SemaphoreType.DMA, pltpu.SemaphoreType.DMA)
    plsc.subcore_barrier()

    def emit_blk(j, carry):
        base = sid * RPS + j * C
        pltpu.sync_copy(sh_agg.at[pl.ds(base, C)], rows_v)
        pltpu.sync_copy(rows_v, out_agg.at[cid, pl.ds(base, C)])
        return carry

    lax.fori_loop(0, RPS // C, emit_blk, 0)


def _make_sc_agg(const_rows):
    scratch = []
    if not const_rows:
        scratch.append(pltpu.VMEM((IB, C), jnp.int32))
    scratch.extend([
        pltpu.VMEM((IB, C), jnp.int32),
        pltpu.VMEM((C, D), jnp.float32),
        pltpu.VMEM((C, D), jnp.float32),
        pltpu.VMEM_SHARED((NP, D), jnp.float32),
    ])
    return pl.kernel(
        functools.partial(_sc_agg_body, const_rows),
        out_type=[jax.ShapeDtypeStruct((NC, NP, D), jnp.float32)],
        mesh=plsc.VectorSubcoreMesh(core_axis_name="c", subcore_axis_name="s"),
        scratch_types=scratch,
    )


def _tc_layer_body(x_ref, p_ref, d_ref, w_ref, b_ref, o_ref):
    deg = d_ref[0, :, :1] + d_ref[1, :, :1]
    inv = 1.0 / jnp.maximum(deg, 1.0)
    agg = (p_ref[0] + p_ref[1]) * inv
    acc = jnp.dot(x_ref[...], w_ref[:D], preferred_element_type=jnp.float32)
    acc = acc + jnp.dot(agg, w_ref[D:], preferred_element_type=jnp.float32)
    o_ref[...] = jnp.maximum(acc + b_ref[...], 0.0)


def _tc_layer(x, parts, degp, W, b2d, blk=2000):
    grid = (N // blk,)
    return pl.pallas_call(
        _tc_layer_body,
        grid=grid,
        in_specs=[
            pl.BlockSpec((blk, D), lambda i: (i, 0)),
            pl.BlockSpec((NC, blk, D), lambda i: (0, i, 0)),
            pl.BlockSpec((NC, blk, D), lambda i: (0, i, 0)),
            pl.BlockSpec((2 * D, D), lambda i: (0, 0)),
            pl.BlockSpec((1, D), lambda i: (0, 0)),
        ],
        out_specs=pl.BlockSpec((blk, D), lambda i: (i, 0)),
        out_shape=jax.ShapeDtypeStruct((N, D), jnp.float32),
    )(x, parts, degp, W, b2d)


_sc_agg = _make_sc_agg(False)
_sc_deg = _make_sc_agg(True)


@jax.jit
def kernel(x, edge_index, W1, b1, W2, b2):
    src = edge_index[0].reshape(NW, G, IB, C)
    dst = edge_index[1].reshape(NW, G, IB, C)
    z2d = jnp.zeros((C, D), jnp.float32)
    ones_cd = jnp.ones((C, D), jnp.float32)

    (degp,) = _sc_deg(ones_cd, dst, z2d)
    (parts1,) = _sc_agg(x, src, dst, z2d)
    h = _tc_layer(x, parts1, degp, W1, b1.reshape(1, D))
    (parts2,) = _sc_agg(h, src, dst, z2d)
    return _tc_layer(h, parts2, degp, W2, b2.reshape(1, D))


# speedup vs baseline: 3.2645x; 1.3096x over previous
import functools

import jax
import jax.numpy as jnp
from jax import lax
from jax.experimental import pallas as pl
from jax.experimental.pallas import tpu as pltpu
from jax.experimental.pallas import tpu_sc as plsc

N = 10000
E = 320000
D = 128

NC = 2
NS = 16
NW = NC * NS
C = 80
G = 5
IB = 25
CH = G * IB
NP = 10240
RPS = NP // NS


def _sc_agg_body(const_rows, *refs):
    if const_rows:
        (ones_cd, dstr, z2d, out_agg, dst_v, rows_v, sh_agg) = refs
    else:
        (feat, srcr, dstr, z2d, out_agg,
         src_v, dst_v, rows_v, rows_v1, rows_v2, sh_agg) = refs
        bufs = (rows_v, rows_v1, rows_v2)

    cid = lax.axis_index("c")
    sid = lax.axis_index("s")
    wid = sid * NC + cid

    pltpu.sync_copy(z2d, rows_v)

    def zero_blk(j, carry):
        pltpu.sync_copy(rows_v, sh_agg.at[pl.ds(sid * RPS + j * C, C)])
        return carry

    lax.fori_loop(0, RPS // C, zero_blk, 0)
    if const_rows:
        pltpu.sync_copy(ones_cd, rows_v)
    plsc.subcore_barrier()

    def pipeline(sem_g, sem_s):
        def superchunk(g, carry):
            if not const_rows:
                pltpu.sync_copy(srcr.at[wid, g], src_v)
            pltpu.sync_copy(dstr.at[wid, g], dst_v)

            if const_rows:
                cps = [
                    pltpu.async_copy(
                        rows_v, sh_agg.at[dst_v.at[k]], sem_s, add=True)
                    for k in range(IB)
                ]
                for cp in cps:
                    cp.wait()
                return carry

            nb = len(bufs)
            gathers = [None] * IB
            scatters = [None] * IB
            for k in range(min(nb - 1, IB)):
                gathers[k] = pltpu.async_copy(
                    feat.at[src_v.at[k]], bufs[k % nb], sem_g)
            for k in range(IB):
                gathers[k].wait()
                if k >= 1:
                    scatters[k - 1].wait()
                if k + nb - 1 < IB:
                    gathers[k + nb - 1] = pltpu.async_copy(
                        feat.at[src_v.at[k + nb - 1]],
                        bufs[(k + nb - 1) % nb], sem_g)
                scatters[k] = pltpu.async_copy(
                    bufs[k % nb], sh_agg.at[dst_v.at[k]], sem_s, add=True)
            scatters[IB - 1].wait()
            return carry

        lax.fori_loop(0, G, superchunk, 0)

    pl.run_scoped(pipeline, pltpu.SemaphoreType.DMA, pltpu.SemaphoreType.DMA)
    plsc.subcore_barrier()

    def emit_blk(j, carry):
        base = sid * RPS + j * C
        pltpu.sync_copy(sh_agg.at[pl.ds(base, C)], rows_v)
        pltpu.sync_copy(rows_v, out_agg.at[cid, pl.ds(base, C)])
        return carry

    lax.fori_loop(0, RPS // C, emit_blk, 0)


def _make_sc_agg(const_rows):
    scratch = []
    if not const_rows:
        scratch.append(pltpu.VMEM((IB, C), jnp.int32))
    scratch.extend([
        pltpu.VMEM((IB, C), jnp.int32),
        pltpu.VMEM((C, D), jnp.float32),
        pltpu.VMEM_SHARED((NP, D), jnp.float32),
    ])
    if not const_rows:
        scratch.insert(-1, pltpu.VMEM((C, D), jnp.float32))
        scratch.insert(-1, pltpu.VMEM((C, D), jnp.float32))
    return pl.kernel(
        functools.partial(_sc_agg_body, const_rows),
        out_type=[jax.ShapeDtypeStruct((NC, NP, D), jnp.float32)],
        mesh=plsc.VectorSubcoreMesh(core_axis_name="c", subcore_axis_name="s"),
        scratch_types=scratch,
    )


def _tc_layer_body(x_ref, p_ref, d_ref, w_ref, b_ref, o_ref):
    deg = d_ref[0, :, :1] + d_ref[1, :, :1]
    inv = 1.0 / jnp.maximum(deg, 1.0)
    agg = (p_ref[0] + p_ref[1]) * inv
    acc = jnp.dot(x_ref[...], w_ref[:D], preferred_element_type=jnp.float32)
    acc = acc + jnp.dot(agg, w_ref[D:], preferred_element_type=jnp.float32)
    o_ref[...] = jnp.maximum(acc + b_ref[...], 0.0)


def _tc_layer(x, parts, degp, W, b2d, blk=2000):
    grid = (N // blk,)
    return pl.pallas_call(
        _tc_layer_body,
        grid=grid,
        in_specs=[
            pl.BlockSpec((blk, D), lambda i: (i, 0)),
            pl.BlockSpec((NC, blk, D), lambda i: (0, i, 0)),
            pl.BlockSpec((NC, blk, D), lambda i: (0, i, 0)),
            pl.BlockSpec((2 * D, D), lambda i: (0, 0)),
            pl.BlockSpec((1, D), lambda i: (0, 0)),
        ],
        out_specs=pl.BlockSpec((blk, D), lambda i: (i, 0)),
        out_shape=jax.ShapeDtypeStruct((N, D), jnp.float32),
    )(x, parts, degp, W, b2d)


_sc_agg = _make_sc_agg(False)
_sc_deg = _make_sc_agg(True)


@jax.jit
def kernel(x, edge_index, W1, b1, W2, b2):
    src = edge_index[0].reshape(NW, G, IB, C)
    dst = edge_index[1].reshape(NW, G, IB, C)
    z2d = jnp.zeros((C, D), jnp.float32)
    ones_cd = jnp.ones((C, D), jnp.float32)

    (degp,) = _sc_deg(ones_cd, dst, z2d)
    (parts1,) = _sc_agg(x, src, dst, z2d)
    h = _tc_layer(x, parts1, degp, W1, b1.reshape(1, D))
    (parts2,) = _sc_agg(h, src, dst, z2d)
    return _tc_layer(h, parts2, degp, W2, b2.reshape(1, D))
